# baseline (speedup 1.0000x reference)
import jax, jax.numpy as jnp
from jax.experimental import pallas as pl
from jax.experimental.pallas import tpu as pltpu

_BM1 = 256
_CPARAMS = pltpu.CompilerParams(
    dimension_semantics=("arbitrary",),
    vmem_limit_bytes=100 * 1024 * 1024,
)

def _p1_body(meta_ref, af_ref, as_ref, ca_ref):
    m = meta_ref[0]
    ca_ref[...] = (m * af_ref[...] + (1.0 - m) * as_ref[...]).astype(jnp.bfloat16)

def kernel(z_feature, z_spatial, adj_feature, adj_spatial, x, meta, W1, W2, W3, wl_W, wl_b, mlp_W, mlp_b):
    bf = jnp.bfloat16
    n = adj_feature.shape[0]
    g1 = pl.cdiv(n, _BM1)
    con_adj = pl.pallas_call(
        _p1_body,
        grid=(g1,),
        in_specs=[
            pl.BlockSpec(memory_space=pltpu.SMEM),
            pl.BlockSpec((_BM1, n), lambda i: (i, 0)),
            pl.BlockSpec((_BM1, n), lambda i: (i, 0)),
        ],
        out_specs=pl.BlockSpec((_BM1, n), lambda i: (i, 0)),
        out_shape=jax.ShapeDtypeStruct((n, n), jnp.bfloat16),
        compiler_params=_CPARAMS,
    )(meta, adj_feature, adj_spatial)

    bdot = lambda a, b: jnp.dot(a.astype(bf), b.astype(bf), preferred_element_type=jnp.float32)
    z = jnp.dot(con_adj, jnp.tanh(x @ W1).astype(bf), preferred_element_type=jnp.float32)
    z = jnp.dot(con_adj, jnp.tanh(z @ W2).astype(bf), preferred_element_type=jnp.float32)
    com = jnp.dot(con_adj, (z @ W3).astype(bf), preferred_element_type=jnp.float32)
    emb = jnp.stack([z_feature, com, z_spatial], axis=1)
    a = emb @ wl_W + wl_b
    nrm = jnp.sqrt(jnp.sum(a * a, axis=1, keepdims=True))
    a = a / jnp.maximum(nrm, 1e-12)
    emb = jnp.concatenate([a[:, 0] * z_feature, a[:, 1] * com, a[:, 2] * z_spatial], axis=1)
    return emb @ mlp_W + mlp_b


# speedup vs baseline: 1.0009x; 1.0009x over previous
import jax
import jax.numpy as jnp
from jax.experimental import pallas as pl
from jax.experimental.pallas import tpu as pltpu

_BM1 = 256
_CPARAMS = pltpu.CompilerParams(
    dimension_semantics=("arbitrary",),
    vmem_limit_bytes=100 * 1024 * 1024,
)


def _p1_body(meta_ref, af_ref, as_ref, ca_ref):
    m = meta_ref[0]
    ca_ref[...] = (m * af_ref[...] + (1.0 - m) * as_ref[...]).astype(jnp.bfloat16)


def kernel(z_feature, z_spatial, adj_feature, adj_spatial, x,
           meta, W1, W2, W3, wl_W, wl_b, mlp_W, mlp_b):
    bf = jnp.bfloat16
    n = adj_feature.shape[0]
    g1 = pl.cdiv(n, _BM1)
    con_adj = pl.pallas_call(
        _p1_body,
        grid=(g1,),
        in_specs=[
            pl.BlockSpec(memory_space=pltpu.SMEM),
            pl.BlockSpec((_BM1, n), lambda i: (i, 0)),
            pl.BlockSpec((_BM1, n), lambda i: (i, 0)),
        ],
        out_specs=pl.BlockSpec((_BM1, n), lambda i: (i, 0)),
        out_shape=jax.ShapeDtypeStruct((n, n), jnp.bfloat16),
        compiler_params=_CPARAMS,
    )(meta, adj_feature, adj_spatial)

    z = jnp.dot(con_adj, jnp.tanh(x @ W1).astype(bf),
                preferred_element_type=jnp.float32)
    z = jnp.dot(con_adj, jnp.tanh(z @ W2).astype(bf),
                preferred_element_type=jnp.float32)
    com = jnp.dot(con_adj, (z @ W3).astype(bf),
                  preferred_element_type=jnp.float32)
    emb = jnp.stack([z_feature, com, z_spatial], axis=1)
    a = emb @ wl_W + wl_b
    nrm = jnp.sqrt(jnp.sum(a * a, axis=1, keepdims=True))
    a = a / jnp.maximum(nrm, 1e-12)
    emb = jnp.concatenate(
        [a[:, 0] * z_feature, a[:, 1] * com, a[:, 2] * z_spatial], axis=1)
    return emb @ mlp_W + mlp_b
